# baseline (speedup 1.0000x reference)
import functools

import jax
import jax.numpy as jnp
from jax import lax
from jax.experimental import pallas as pl
from jax.experimental.pallas import tpu as pltpu
from jax.experimental.pallas import tpu_sc as plsc

N = 4096
E = 65536
HID = 256
ZD = 64
HEADS = 4
NW = 32
EPW = E // NW
NCH = 16
CH = EPW // NCH
W128 = 128

_mesh = plsc.VectorSubcoreMesh(core_axis_name="c", subcore_axis_name="s")


@functools.partial(
    pl.kernel, mesh=_mesh,
    out_type=jax.ShapeDtypeStruct((2 * N, W128), jnp.float32),
    scratch_types=[
        pltpu.VMEM((NCH, CH), jnp.int32),
        pltpu.VMEM((CH, W128), jnp.float32),
        pltpu.VMEM_SHARED((N, W128), jnp.float32),
    ],
)
def _sc_deg(colr, ones, zeros, out, cidx, ones_v, acc):
    cid = lax.axis_index("c")
    sid = lax.axis_index("s")
    wid = cid * 16 + sid
    pltpu.sync_copy(zeros, acc.at[pl.ds(sid * 256, 256)])
    pltpu.sync_copy(colr.at[wid], cidx)
    pltpu.sync_copy(ones, ones_v)
    plsc.subcore_barrier()
    for j in range(NCH):
        pltpu.sync_copy(ones_v, acc.at[cidx.at[j]], add=True)
    plsc.subcore_barrier()
    pltpu.sync_copy(acc.at[pl.ds(sid * 256, 256)],
                    out.at[pl.ds(cid * N + sid * 256, 256)])


@functools.partial(
    pl.kernel, mesh=_mesh,
    out_type=jax.ShapeDtypeStruct((2 * 2 * N, W128), jnp.float32),
    scratch_types=[
        pltpu.VMEM((NCH, CH), jnp.int32),
        pltpu.VMEM((NCH, CH), jnp.int32),
        pltpu.VMEM((CH, W128), jnp.float32),
        pltpu.VMEM((CH, W128), jnp.float32),
        pltpu.VMEM((CH, W128), jnp.float32),
        pltpu.VMEM_SHARED((N, W128), jnp.float32),
        pltpu.SemaphoreType.DMA,
        pltpu.SemaphoreType.DMA,
        pltpu.SemaphoreType.DMA,
        pltpu.SemaphoreType.DMA,
        pltpu.SemaphoreType.DMA,
        pltpu.SemaphoreType.DMA,
    ],
)
def _sc_gcn(xwpL, xwpR, rowr, colr, zeros, out, ridx, cidx, pay0, pay1, pay2,
            acc, g0, g1, g2, s0, s1, s2):
    cid = lax.axis_index("c")
    sid = lax.axis_index("s")
    wid = cid * 16 + sid
    pltpu.sync_copy(zeros, acc.at[pl.ds(sid * 256, 256)])
    pltpu.sync_copy(rowr.at[wid], ridx)
    pltpu.sync_copy(colr.at[wid], cidx)
    plsc.subcore_barrier()
    pays = (pay0, pay1, pay2)
    gsems = (g0, g1, g2)
    ssems = (s0, s1, s2)
    sps = [None, None, None]
    for t, tab in enumerate((xwpL, xwpR)):
        cps = [None, None, None]
        cps[0] = pltpu.async_copy(tab.at[ridx.at[0]], pays[0], gsems[0])
        for j in range(NCH):
            b = j % 3
            if j + 1 < NCH:
                nb = (j + 1) % 3
                if sps[nb] is not None:
                    sps[nb].wait()
                    sps[nb] = None
                cps[nb] = pltpu.async_copy(tab.at[ridx.at[j + 1]], pays[nb], gsems[nb])
            cps[b].wait()
            sps[b] = pltpu.async_copy(pays[b], acc.at[cidx.at[j]], ssems[b],
                                      add=True)
        for b in range(3):
            if sps[b] is not None:
                sps[b].wait()
                sps[b] = None
        plsc.subcore_barrier()
        pltpu.sync_copy(acc.at[pl.ds(sid * 256, 256)],
                        out.at[pl.ds((cid * 2 + t) * N + sid * 256, 256)])
        if t == 0:
            pltpu.sync_copy(zeros, acc.at[pl.ds(sid * 256, 256)])
        plsc.subcore_barrier()


@functools.partial(
    pl.kernel, mesh=_mesh,
    out_type=jax.ShapeDtypeStruct((2 * 9 * N, W128), jnp.float32),
    scratch_types=[
        pltpu.VMEM((NCH, CH), jnp.int32),
        pltpu.VMEM((NCH, CH), jnp.int32),
        pltpu.VMEM((CH, W128), jnp.float32),
        pltpu.VMEM((CH, W128), jnp.float32),
        pltpu.VMEM((CH, W128), jnp.float32),
        pltpu.VMEM((NCH, CH * 16), jnp.float32),
        pltpu.VMEM_SHARED((N, W128), jnp.float32),
        pltpu.SemaphoreType.DMA,
        pltpu.SemaphoreType.DMA,
        pltpu.SemaphoreType.DMA,
        pltpu.SemaphoreType.DMA,
        pltpu.SemaphoreType.DMA,
        pltpu.SemaphoreType.DMA,
    ],
)
def _sc_gat(t0, t1, t2, t3, t4, t5, t6, t7, rowr, colr, atab, zeros,
            out, ridx, cidx, pay0, pay1, pay2, pbuf, acc, g0, g1, g2,
            s0, s1, s2):
    cid = lax.axis_index("c")
    sid = lax.axis_index("s")
    wid = cid * 16 + sid
    pltpu.sync_copy(zeros, acc.at[pl.ds(sid * 256, 256)])
    pltpu.sync_copy(rowr.at[wid], ridx)
    pltpu.sync_copy(colr.at[wid], cidx)

    def wait_gather(pay, sem):
        pltpu.make_async_copy(atab.at[ridx.at[0]], pay, sem).wait()

    def wait_scatter(pay, sem):
        pltpu.make_async_copy(pay, acc.at[cidx.at[0]], sem).wait()

    def p1_body(j, _):
        cp0 = pltpu.async_copy(atab.at[ridx.at[j]], pay0, g0)
        cp1 = pltpu.async_copy(atab.at[cidx.at[j]], pay1, g1)
        cp0.wait()
        cp1.wait()

        def p_body(e, _):
            ev = pay0[e, pl.ds(0, 16)] + pay1[e, pl.ds(16, 16)]
            ev = jnp.maximum(ev, 0.0) + 0.2 * jnp.minimum(ev, 0.0)
            pbuf[j, pl.ds(e * 16, 16)] = jnp.exp(ev - pay1[e, pl.ds(32, 16)])
            return 0

        lax.fori_loop(0, CH, p_body, 0)
        return 0

    lax.fori_loop(0, NCH, p1_body, 0)

    plsc.subcore_barrier()

    tabs = (t0, t1, t2, t3, t4, t5, t6, t7)
    for t in range(8):
        h = t // 2
        tab = tabs[t]
        pltpu.async_copy(tab.at[ridx.at[0]], pay0, g0)

        def scale_scatter(pay, gsem, ssem, other_pay, other_ssem, j):
            wait_gather(pay, gsem)

            def s_body(e4, _):
                for u in range(4):
                    e = e4 * 4 + u
                    ps = pbuf[j, pl.ds(e * 16, 16)][h]
                    for c in range(W128 // 16):
                        pay[e, pl.ds(c * 16, 16)] = pay[e, pl.ds(c * 16, 16)] * ps
                return 0

            lax.fori_loop(0, CH // 4, s_body, 0)
            pltpu.async_copy(pay, acc.at[cidx.at[j]], ssem, add=True)

        def j_body(j, _):
            even = j % 2 == 0

            @pl.when(jnp.logical_and(j > 0, j < NCH - 1))
            def _():
                @pl.when(even)
                def _():
                    @pl.when(j > 1)
                    def _():
                        wait_scatter(pay1, s1)
                    pltpu.async_copy(tab.at[ridx.at[j + 1]], pay1, g1)

                @pl.when(jnp.logical_not(even))
                def _():
                    wait_scatter(pay0, s0)
                    pltpu.async_copy(tab.at[ridx.at[j + 1]], pay0, g0)

            @pl.when(jnp.logical_and(j == 0, NCH > 1))
            def _():
                pltpu.async_copy(tab.at[ridx.at[1]], pay1, g1)

            @pl.when(even)
            def _():
                scale_scatter(pay0, g0, s0, pay1, s1, j)

            @pl.when(jnp.logical_not(even))
            def _():
                scale_scatter(pay1, g1, s1, pay0, s0, j)

            return 0

        lax.fori_loop(0, NCH, j_body, 0)
        wait_scatter(pay0, s0)
        wait_scatter(pay1, s1)
        plsc.subcore_barrier()
        pltpu.sync_copy(acc.at[pl.ds(sid * 256, 256)],
                        out.at[pl.ds((cid * 9 + t) * N + sid * 256, 256)])
        pltpu.sync_copy(zeros, acc.at[pl.ds(sid * 256, 256)])
        plsc.subcore_barrier()

    def p3_body(j, _):
        even = j % 2 == 0

        def build_scatter(pay, ssem):
            def d_body(e, _):
                pv = pbuf[j, pl.ds(e * 16, 16)]
                for c in range(W128 // 16):
                    pay[e, pl.ds(c * 16, 16)] = pv
                return 0

            lax.fori_loop(0, CH, d_body, 0)
            pltpu.async_copy(pay, acc.at[cidx.at[j]], ssem, add=True)

        @pl.when(even)
        def _():
            @pl.when(j > 1)
            def _():
                wait_scatter(pay0, s0)
            build_scatter(pay0, s0)

        @pl.when(jnp.logical_not(even))
        def _():
            @pl.when(j > 1)
            def _():
                wait_scatter(pay1, s1)
            build_scatter(pay1, s1)

        return 0

    lax.fori_loop(0, NCH, p3_body, 0)
    wait_scatter(pay0, s0)
    wait_scatter(pay1, s1)
    plsc.subcore_barrier()
    pltpu.sync_copy(acc.at[pl.ds(sid * 256, 256)],
                    out.at[pl.ds((cid * 9 + 8) * N + sid * 256, 256)])


_BR = 512


def _tc_a_body(x_ref, w_ref, degp_ref, xw_ref, xwpL_ref, xwpR_ref):
    xw = jnp.dot(x_ref[...], w_ref[...], preferred_element_type=jnp.float32)
    deg = degp_ref[0, :, 0] + degp_ref[1, :, 0] + 1.0
    dinv = lax.rsqrt(deg)
    xw_ref[...] = xw
    xwp = xw * dinv[:, None]
    xwpL_ref[...] = xwp[:, :W128]
    xwpR_ref[...] = xwp[:, W128:]


def _tc_b_body(xw_ref, degp_ref, aggp_ref, gcnb_ref, gatW_ref, atts_ref, attd_ref,
               t0_ref, t1_ref, t2_ref, t3_ref, t4_ref, t5_ref, t6_ref, t7_ref,
               asrc_ref, adst_ref, amax_ref):
    i = pl.program_id(0)
    deg = degp_ref[0, :, 0] + degp_ref[1, :, 0] + 1.0
    dinv = lax.rsqrt(deg)
    agghalves = aggp_ref[0] + aggp_ref[1]
    agg = jnp.concatenate([agghalves[0], agghalves[1]], axis=-1)
    xw = xw_ref[...]
    h = jax.nn.relu(dinv[:, None] * agg + (dinv * dinv)[:, None] * xw + gcnb_ref[...])
    xw4 = jnp.dot(h, gatW_ref[...], preferred_element_type=jnp.float32)
    trefs = (t0_ref, t1_ref, t2_ref, t3_ref, t4_ref, t5_ref, t6_ref, t7_ref)
    for t in range(8):
        trefs[t][...] = xw4[:, t * W128:(t + 1) * W128]
    xw4r = xw4.reshape(_BR, HEADS, HID)
    asrc = jnp.sum(xw4r * atts_ref[...][None], axis=-1)
    adst = jnp.sum(xw4r * attd_ref[...][None], axis=-1)
    asrc_ref[...] = asrc
    adst_ref[...] = adst

    @pl.when(i == 0)
    def _():
        amax_ref[...] = jnp.full((1, HEADS), -jnp.inf, jnp.float32)

    amax_ref[...] = jnp.maximum(amax_ref[...], jnp.max(asrc, axis=0, keepdims=True))


def _lrelu(x):
    return jnp.maximum(x, 0.0) + 0.2 * jnp.minimum(x, 0.0)


def _tc_b2_body(asrc_ref, adst_ref, amax_ref, atab_ref, pself_ref):
    asrc = asrc_ref[...]
    adst = adst_ref[...]
    amax = amax_ref[...]
    m = _lrelu(amax + adst)
    eself = _lrelu(asrc + adst)
    pself_ref[...] = jnp.exp(eself - m)
    atab_ref[...] = jnp.concatenate(
        [jnp.tile(asrc, (1, 4)), jnp.tile(adst, (1, 4)), jnp.tile(m, (1, 4)),
         jnp.zeros((N, 80), jnp.float32)], axis=-1)


def _tc_c_body(outp_ref, t0_ref, t1_ref, t2_ref, t3_ref, t4_ref, t5_ref,
               t6_ref, t7_ref, pself_ref, eps_ref,
               gatb_ref, w1_ref, b1_ref, w2_ref, b2_ref, muW_ref, mub_ref,
               lvW_ref, lvb_ref, dW1_ref, db1_ref, dW2_ref, db2_ref,
               mu_ref, lv_ref, hd_ref):
    pre = outp_ref[0] + outp_ref[1]
    pself = pself_ref[...]
    trefs = (t0_ref, t1_ref, t2_ref, t3_ref, t4_ref, t5_ref, t6_ref, t7_ref)
    acc = jnp.zeros((_BR, HID), jnp.float32)
    for hh in range(HEADS):
        feat = jnp.concatenate([pre[2 * hh], pre[2 * hh + 1]], axis=-1)
        xw4h = jnp.concatenate([trefs[2 * hh][...], trefs[2 * hh + 1][...]], axis=-1)
        ph = pself[:, hh][:, None]
        feat = feat + ph * xw4h
        s = pre[8][:, hh][:, None] + ph
        acc = acc + feat / (s + 1e-16)
    hg = jax.nn.relu(acc * (1.0 / HEADS) + gatb_ref[...])
    h2 = jax.nn.relu(jnp.dot(hg, w1_ref[...], preferred_element_type=jnp.float32) + b1_ref[...])
    h3 = jax.nn.relu(jnp.dot(h2, w2_ref[...], preferred_element_type=jnp.float32) + b2_ref[...])
    mu = jnp.dot(h3, muW_ref[...], preferred_element_type=jnp.float32) + mub_ref[...]
    logvar = jnp.dot(h3, lvW_ref[...], preferred_element_type=jnp.float32) + lvb_ref[...]
    lvc = jnp.clip(logvar, -10.0, 10.0)
    z = mu + eps_ref[...] * jnp.exp(0.5 * lvc)
    hd1 = jax.nn.relu(jnp.dot(z, dW1_ref[...], preferred_element_type=jnp.float32) + db1_ref[...])
    hd = jnp.dot(hd1, dW2_ref[...], preferred_element_type=jnp.float32) + db2_ref[...] + z
    mu_ref[...] = mu
    lv_ref[...] = logvar
    hd_ref[...] = hd


def _tc_d_body(hdb_ref, hdf_ref, out_ref):
    out_ref[...] = lax.dot_general(
        hdb_ref[...], hdf_ref[...], (((1,), (1,)), ((), ())),
        preferred_element_type=jnp.float32)


def kernel(x, edge_index, eps, gcn_W, gcn_b, gat_W, gat_att_src, gat_att_dst,
           gat_b, mlp_W1, mlp_b1, mlp_W2, mlp_b2, mu_W, mu_b, lv_W, lv_b,
           dec_W1, dec_b1, dec_W2, dec_b2):
    f32 = jnp.float32
    row = edge_index[0].astype(jnp.int32).reshape(NW, NCH, CH)
    col = edge_index[1].astype(jnp.int32).reshape(NW, NCH, CH)
    ones128 = jnp.ones((CH, W128), f32)
    zeros128 = jnp.zeros((256, W128), f32)

    degp = _sc_deg(col, ones128, zeros128)
    degp2 = degp.reshape(2, N, W128)

    nb = 8
    xw, xwpL, xwpR = pl.pallas_call(
        _tc_a_body,
        grid=(nb,),
        in_specs=[
            pl.BlockSpec((_BR, 256), lambda i: (i, 0)),
            pl.BlockSpec((256, HID), lambda i: (0, 0)),
            pl.BlockSpec((2, _BR, W128), lambda i: (0, i, 0)),
        ],
        out_specs=[
            pl.BlockSpec((_BR, HID), lambda i: (i, 0)),
            pl.BlockSpec((_BR, W128), lambda i: (i, 0)),
            pl.BlockSpec((_BR, W128), lambda i: (i, 0)),
        ],
        out_shape=[
            jax.ShapeDtypeStruct((N, HID), f32),
            jax.ShapeDtypeStruct((N, W128), f32),
            jax.ShapeDtypeStruct((N, W128), f32),
        ],
    )(x, gcn_W, degp2)

    aggp = _sc_gcn(xwpL, xwpR, row, col, zeros128).reshape(2, 2, N, W128)

    t_shape = jax.ShapeDtypeStruct((N, W128), f32)
    t_spec = pl.BlockSpec((_BR, W128), lambda i: (i, 0))
    outs_b = pl.pallas_call(
        _tc_b_body,
        grid=(nb,),
        in_specs=[
            pl.BlockSpec((_BR, HID), lambda i: (i, 0)),
            pl.BlockSpec((2, _BR, W128), lambda i: (0, i, 0)),
            pl.BlockSpec((2, 2, _BR, W128), lambda i: (0, 0, i, 0)),
            pl.BlockSpec((1, HID), lambda i: (0, 0)),
            pl.BlockSpec((HID, HEADS * HID), lambda i: (0, 0)),
            pl.BlockSpec((HEADS, HID), lambda i: (0, 0)),
            pl.BlockSpec((HEADS, HID), lambda i: (0, 0)),
        ],
        out_specs=[t_spec] * 8 + [
            pl.BlockSpec((_BR, HEADS), lambda i: (i, 0)),
            pl.BlockSpec((_BR, HEADS), lambda i: (i, 0)),
            pl.BlockSpec((1, HEADS), lambda i: (0, 0)),
        ],
        out_shape=[t_shape] * 8 + [
            jax.ShapeDtypeStruct((N, HEADS), f32),
            jax.ShapeDtypeStruct((N, HEADS), f32),
            jax.ShapeDtypeStruct((1, HEADS), f32),
        ],
    )(xw, degp2, aggp, gcn_b.reshape(1, HID), gat_W, gat_att_src, gat_att_dst)
    tbls = outs_b[:8]
    asrc, adst, amax = outs_b[8:]

    atab, pself = pl.pallas_call(
        _tc_b2_body,
        out_shape=[
            jax.ShapeDtypeStruct((N, W128), f32),
            jax.ShapeDtypeStruct((N, HEADS), f32),
        ],
    )(asrc, adst, amax)

    outp = _sc_gat(*tbls, row, col, atab, zeros128)
    outp = outp.reshape(2, 9, N, W128)

    mu, logvar, hd = pl.pallas_call(
        _tc_c_body,
        grid=(nb,),
        in_specs=[
            pl.BlockSpec((2, 9, _BR, W128), lambda i: (0, 0, i, 0)),
        ] + [t_spec] * 8 + [
            pl.BlockSpec((_BR, HEADS), lambda i: (i, 0)),
            pl.BlockSpec((_BR, ZD), lambda i: (i, 0)),
            pl.BlockSpec((1, HID), lambda i: (0, 0)),
            pl.BlockSpec((HID, HID), lambda i: (0, 0)),
            pl.BlockSpec((1, HID), lambda i: (0, 0)),
            pl.BlockSpec((HID, HID), lambda i: (0, 0)),
            pl.BlockSpec((1, HID), lambda i: (0, 0)),
            pl.BlockSpec((HID, ZD), lambda i: (0, 0)),
            pl.BlockSpec((1, ZD), lambda i: (0, 0)),
            pl.BlockSpec((HID, ZD), lambda i: (0, 0)),
            pl.BlockSpec((1, ZD), lambda i: (0, 0)),
            pl.BlockSpec((ZD, HID), lambda i: (0, 0)),
            pl.BlockSpec((1, HID), lambda i: (0, 0)),
            pl.BlockSpec((HID, ZD), lambda i: (0, 0)),
            pl.BlockSpec((1, ZD), lambda i: (0, 0)),
        ],
        out_specs=[
            pl.BlockSpec((_BR, ZD), lambda i: (i, 0)),
            pl.BlockSpec((_BR, ZD), lambda i: (i, 0)),
            pl.BlockSpec((_BR, ZD), lambda i: (i, 0)),
        ],
        out_shape=[
            jax.ShapeDtypeStruct((N, ZD), f32),
            jax.ShapeDtypeStruct((N, ZD), f32),
            jax.ShapeDtypeStruct((N, ZD), f32),
        ],
    )(outp, *tbls, pself, eps,
      gat_b.reshape(1, HID), mlp_W1, mlp_b1.reshape(1, HID),
      mlp_W2, mlp_b2.reshape(1, HID), mu_W, mu_b.reshape(1, ZD),
      lv_W, lv_b.reshape(1, ZD), dec_W1, dec_b1.reshape(1, HID),
      dec_W2, dec_b2.reshape(1, ZD))

    logits = pl.pallas_call(
        _tc_d_body,
        grid=(nb,),
        in_specs=[
            pl.BlockSpec((_BR, ZD), lambda i: (i, 0)),
            pl.BlockSpec((N, ZD), lambda i: (0, 0)),
        ],
        out_specs=pl.BlockSpec((_BR, N), lambda i: (i, 0)),
        out_shape=jax.ShapeDtypeStruct((N, N), f32),
    )(hd, hd)

    return logits, mu, logvar


# speedup vs baseline: 1.0012x; 1.0012x over previous
import functools

import jax
import jax.numpy as jnp
from jax import lax
from jax.experimental import pallas as pl
from jax.experimental.pallas import tpu as pltpu
from jax.experimental.pallas import tpu_sc as plsc

N = 4096
E = 65536
HID = 256
ZD = 64
HEADS = 4
NW = 32
EPW = E // NW
NCH = 16
CH = EPW // NCH
W128 = 128

_mesh = plsc.VectorSubcoreMesh(core_axis_name="c", subcore_axis_name="s")


@functools.partial(
    pl.kernel, mesh=_mesh,
    out_type=jax.ShapeDtypeStruct((2 * N, W128), jnp.float32),
    scratch_types=[
        pltpu.VMEM((NCH, CH), jnp.int32),
        pltpu.VMEM((CH, W128), jnp.float32),
        pltpu.VMEM_SHARED((N, W128), jnp.float32),
    ],
)
def _sc_deg(colr, ones, zeros, out, cidx, ones_v, acc):
    cid = lax.axis_index("c")
    sid = lax.axis_index("s")
    wid = cid * 16 + sid
    pltpu.sync_copy(zeros, acc.at[pl.ds(sid * 256, 256)])
    pltpu.sync_copy(colr.at[wid], cidx)
    pltpu.sync_copy(ones, ones_v)
    plsc.subcore_barrier()
    for j in range(NCH):
        pltpu.sync_copy(ones_v, acc.at[cidx.at[j]], add=True)
    plsc.subcore_barrier()
    pltpu.sync_copy(acc.at[pl.ds(sid * 256, 256)],
                    out.at[pl.ds(cid * N + sid * 256, 256)])


@functools.partial(
    pl.kernel, mesh=_mesh,
    out_type=jax.ShapeDtypeStruct((2 * 2 * N, W128), jnp.float32),
    scratch_types=[
        pltpu.VMEM((NCH, CH), jnp.int32),
        pltpu.VMEM((NCH, CH), jnp.int32),
        pltpu.VMEM((CH, W128), jnp.float32),
        pltpu.VMEM((CH, W128), jnp.float32),
        pltpu.VMEM((CH, W128), jnp.float32),
        pltpu.VMEM_SHARED((N, W128), jnp.float32),
        pltpu.SemaphoreType.DMA,
        pltpu.SemaphoreType.DMA,
        pltpu.SemaphoreType.DMA,
        pltpu.SemaphoreType.DMA,
        pltpu.SemaphoreType.DMA,
        pltpu.SemaphoreType.DMA,
    ],
)
def _sc_gcn(xwpL, xwpR, rowr, colr, zeros, out, ridx, cidx, pay0, pay1, pay2,
            acc, g0, g1, g2, s0, s1, s2):
    cid = lax.axis_index("c")
    sid = lax.axis_index("s")
    wid = cid * 16 + sid
    pltpu.sync_copy(zeros, acc.at[pl.ds(sid * 256, 256)])
    pltpu.sync_copy(rowr.at[wid], ridx)
    pltpu.sync_copy(colr.at[wid], cidx)
    plsc.subcore_barrier()
    pays = (pay0, pay1, pay2)
    gsems = (g0, g1, g2)
    ssems = (s0, s1, s2)
    sps = [None, None, None]
    for t, tab in enumerate((xwpL, xwpR)):
        cps = [None, None, None]
        cps[0] = pltpu.async_copy(tab.at[ridx.at[0]], pays[0], gsems[0])
        for j in range(NCH):
            b = j % 3
            if j + 1 < NCH:
                nb = (j + 1) % 3
                if sps[nb] is not None:
                    sps[nb].wait()
                    sps[nb] = None
                cps[nb] = pltpu.async_copy(tab.at[ridx.at[j + 1]], pays[nb], gsems[nb])
            cps[b].wait()
            sps[b] = pltpu.async_copy(pays[b], acc.at[cidx.at[j]], ssems[b],
                                      add=True)
        for b in range(3):
            if sps[b] is not None:
                sps[b].wait()
                sps[b] = None
        plsc.subcore_barrier()
        pltpu.sync_copy(acc.at[pl.ds(sid * 256, 256)],
                        out.at[pl.ds((cid * 2 + t) * N + sid * 256, 256)])
        if t == 0:
            pltpu.sync_copy(zeros, acc.at[pl.ds(sid * 256, 256)])
        plsc.subcore_barrier()


@functools.partial(
    pl.kernel, mesh=_mesh,
    out_type=jax.ShapeDtypeStruct((2 * 9 * N, W128), jnp.float32),
    scratch_types=[
        pltpu.VMEM((NCH, CH), jnp.int32),
        pltpu.VMEM((NCH, CH), jnp.int32),
        pltpu.VMEM((CH, W128), jnp.float32),
        pltpu.VMEM((CH, W128), jnp.float32),
        pltpu.VMEM((NCH, CH * 16), jnp.float32),
        pltpu.VMEM_SHARED((N, W128), jnp.float32),
        pltpu.SemaphoreType.DMA,
        pltpu.SemaphoreType.DMA,
        pltpu.SemaphoreType.DMA,
        pltpu.SemaphoreType.DMA,
    ],
)
def _sc_gat(t0, t1, t2, t3, t4, t5, t6, t7, rowr, colr, atab, zeros,
            out, ridx, cidx, pay0, pay1, pbuf, acc, g0, g1,
            s0, s1):
    cid = lax.axis_index("c")
    sid = lax.axis_index("s")
    wid = cid * 16 + sid
    pltpu.sync_copy(zeros, acc.at[pl.ds(sid * 256, 256)])
    pltpu.sync_copy(rowr.at[wid], ridx)
    pltpu.sync_copy(colr.at[wid], cidx)

    def wait_gather(pay, sem):
        pltpu.make_async_copy(atab.at[ridx.at[0]], pay, sem).wait()

    def wait_scatter(pay, sem):
        pltpu.make_async_copy(pay, acc.at[cidx.at[0]], sem).wait()

    def p1_body(j, _):
        cp0 = pltpu.async_copy(atab.at[ridx.at[j]], pay0, g0)
        cp1 = pltpu.async_copy(atab.at[cidx.at[j]], pay1, g1)
        cp0.wait()
        cp1.wait()

        def p_body(e, _):
            ev = pay0[e, pl.ds(0, 16)] + pay1[e, pl.ds(16, 16)]
            ev = jnp.maximum(ev, 0.0) + 0.2 * jnp.minimum(ev, 0.0)
            pbuf[j, pl.ds(e * 16, 16)] = jnp.exp(ev - pay1[e, pl.ds(32, 16)])
            return 0

        lax.fori_loop(0, CH, p_body, 0)
        return 0

    lax.fori_loop(0, NCH, p1_body, 0)

    plsc.subcore_barrier()

    tabs = (t0, t1, t2, t3, t4, t5, t6, t7)
    for t in range(8):
        h = t // 2
        tab = tabs[t]
        pltpu.async_copy(tab.at[ridx.at[0]], pay0, g0)

        def scale_scatter(pay, gsem, ssem, j):
            wait_gather(pay, gsem)

            def s_body(e, _):
                ps = pbuf[j, pl.ds(e * 16, 16)][h]
                for c in range(W128 // 16):
                    pay[e, pl.ds(c * 16, 16)] = pay[e, pl.ds(c * 16, 16)] * ps
                return 0

            lax.fori_loop(0, CH, s_body, 0)
            pltpu.async_copy(pay, acc.at[cidx.at[j]], ssem, add=True)

        def j_body(j, _):
            even = j % 2 == 0

            @pl.when(jnp.logical_and(j > 0, j < NCH - 1))
            def _():
                @pl.when(even)
                def _():
                    @pl.when(j > 1)
                    def _():
                        wait_scatter(pay1, s1)
                    pltpu.async_copy(tab.at[ridx.at[j + 1]], pay1, g1)

                @pl.when(jnp.logical_not(even))
                def _():
                    wait_scatter(pay0, s0)
                    pltpu.async_copy(tab.at[ridx.at[j + 1]], pay0, g0)

            @pl.when(j == 0)
            def _():
                pltpu.async_copy(tab.at[ridx.at[1]], pay1, g1)

            @pl.when(even)
            def _():
                scale_scatter(pay0, g0, s0, j)

            @pl.when(jnp.logical_not(even))
            def _():
                scale_scatter(pay1, g1, s1, j)

            return 0

        lax.fori_loop(0, NCH, j_body, 0)
        wait_scatter(pay0, s0)
        wait_scatter(pay1, s1)
        plsc.subcore_barrier()
        pltpu.sync_copy(acc.at[pl.ds(sid * 256, 256)],
                        out.at[pl.ds((cid * 9 + t) * N + sid * 256, 256)])
        pltpu.sync_copy(zeros, acc.at[pl.ds(sid * 256, 256)])
        plsc.subcore_barrier()

    def p3_body(j, _):
        even = j % 2 == 0

        def build_scatter(pay, ssem):
            def d_body(e, _):
                pv = pbuf[j, pl.ds(e * 16, 16)]
                for c in range(W128 // 16):
                    pay[e, pl.ds(c * 16, 16)] = pv
                return 0

            lax.fori_loop(0, CH, d_body, 0)
            pltpu.async_copy(pay, acc.at[cidx.at[j]], ssem, add=True)

        @pl.when(even)
        def _():
            @pl.when(j > 1)
            def _():
                wait_scatter(pay0, s0)
            build_scatter(pay0, s0)

        @pl.when(jnp.logical_not(even))
        def _():
            @pl.when(j > 1)
            def _():
                wait_scatter(pay1, s1)
            build_scatter(pay1, s1)

        return 0

    lax.fori_loop(0, NCH, p3_body, 0)
    wait_scatter(pay0, s0)
    wait_scatter(pay1, s1)
    plsc.subcore_barrier()
    pltpu.sync_copy(acc.at[pl.ds(sid * 256, 256)],
                    out.at[pl.ds((cid * 9 + 8) * N + sid * 256, 256)])


_BR = 512


def _tc_a_body(x_ref, w_ref, degp_ref, xw_ref, xwpL_ref, xwpR_ref):
    xw = jnp.dot(x_ref[...], w_ref[...], preferred_element_type=jnp.float32)
    deg = degp_ref[0, :, 0] + degp_ref[1, :, 0] + 1.0
    dinv = lax.rsqrt(deg)
    xw_ref[...] = xw
    xwp = xw * dinv[:, None]
    xwpL_ref[...] = xwp[:, :W128]
    xwpR_ref[...] = xwp[:, W128:]


def _tc_b_body(xw_ref, degp_ref, aggp_ref, gcnb_ref, gatW_ref, atts_ref, attd_ref,
               t0_ref, t1_ref, t2_ref, t3_ref, t4_ref, t5_ref, t6_ref, t7_ref,
               asrc_ref, adst_ref, amax_ref):
    i = pl.program_id(0)
    deg = degp_ref[0, :, 0] + degp_ref[1, :, 0] + 1.0
    dinv = lax.rsqrt(deg)
    agghalves = aggp_ref[0] + aggp_ref[1]
    agg = jnp.concatenate([agghalves[0], agghalves[1]], axis=-1)
    xw = xw_ref[...]
    h = jax.nn.relu(dinv[:, None] * agg + (dinv * dinv)[:, None] * xw + gcnb_ref[...])
    xw4 = jnp.dot(h, gatW_ref[...], preferred_element_type=jnp.float32)
    trefs = (t0_ref, t1_ref, t2_ref, t3_ref, t4_ref, t5_ref, t6_ref, t7_ref)
    for t in range(8):
        trefs[t][...] = xw4[:, t * W128:(t + 1) * W128]
    xw4r = xw4.reshape(_BR, HEADS, HID)
    asrc = jnp.sum(xw4r * atts_ref[...][None], axis=-1)
    adst = jnp.sum(xw4r * attd_ref[...][None], axis=-1)
    asrc_ref[...] = asrc
    adst_ref[...] = adst

    @pl.when(i == 0)
    def _():
        amax_ref[...] = jnp.full((1, HEADS), -jnp.inf, jnp.float32)

    amax_ref[...] = jnp.maximum(amax_ref[...], jnp.max(asrc, axis=0, keepdims=True))


def _lrelu(x):
    return jnp.maximum(x, 0.0) + 0.2 * jnp.minimum(x, 0.0)


def _tc_b2_body(asrc_ref, adst_ref, amax_ref, atab_ref, pself_ref):
    asrc = asrc_ref[...]
    adst = adst_ref[...]
    amax = amax_ref[...]
    m = _lrelu(amax + adst)
    eself = _lrelu(asrc + adst)
    pself_ref[...] = jnp.exp(eself - m)
    atab_ref[...] = jnp.concatenate(
        [jnp.tile(asrc, (1, 4)), jnp.tile(adst, (1, 4)), jnp.tile(m, (1, 4)),
         jnp.zeros((N, 80), jnp.float32)], axis=-1)


def _tc_c_body(outp_ref, t0_ref, t1_ref, t2_ref, t3_ref, t4_ref, t5_ref,
               t6_ref, t7_ref, pself_ref, eps_ref,
               gatb_ref, w1_ref, b1_ref, w2_ref, b2_ref, muW_ref, mub_ref,
               lvW_ref, lvb_ref, dW1_ref, db1_ref, dW2_ref, db2_ref,
               mu_ref, lv_ref, hd_ref):
    pre = outp_ref[0] + outp_ref[1]
    pself = pself_ref[...]
    trefs = (t0_ref, t1_ref, t2_ref, t3_ref, t4_ref, t5_ref, t6_ref, t7_ref)
    acc = jnp.zeros((_BR, HID), jnp.float32)
    for hh in range(HEADS):
        feat = jnp.concatenate([pre[2 * hh], pre[2 * hh + 1]], axis=-1)
        xw4h = jnp.concatenate([trefs[2 * hh][...], trefs[2 * hh + 1][...]], axis=-1)
        ph = pself[:, hh][:, None]
        feat = feat + ph * xw4h
        s = pre[8][:, hh][:, None] + ph
        acc = acc + feat / (s + 1e-16)
    hg = jax.nn.relu(acc * (1.0 / HEADS) + gatb_ref[...])
    h2 = jax.nn.relu(jnp.dot(hg, w1_ref[...], preferred_element_type=jnp.float32) + b1_ref[...])
    h3 = jax.nn.relu(jnp.dot(h2, w2_ref[...], preferred_element_type=jnp.float32) + b2_ref[...])
    mu = jnp.dot(h3, muW_ref[...], preferred_element_type=jnp.float32) + mub_ref[...]
    logvar = jnp.dot(h3, lvW_ref[...], preferred_element_type=jnp.float32) + lvb_ref[...]
    lvc = jnp.clip(logvar, -10.0, 10.0)
    z = mu + eps_ref[...] * jnp.exp(0.5 * lvc)
    hd1 = jax.nn.relu(jnp.dot(z, dW1_ref[...], preferred_element_type=jnp.float32) + db1_ref[...])
    hd = jnp.dot(hd1, dW2_ref[...], preferred_element_type=jnp.float32) + db2_ref[...] + z
    mu_ref[...] = mu
    lv_ref[...] = logvar
    hd_ref[...] = hd


def _tc_d_body(hdb_ref, hdf_ref, out_ref):
    out_ref[...] = lax.dot_general(
        hdb_ref[...], hdf_ref[...], (((1,), (1,)), ((), ())),
        preferred_element_type=jnp.float32)


def kernel(x, edge_index, eps, gcn_W, gcn_b, gat_W, gat_att_src, gat_att_dst,
           gat_b, mlp_W1, mlp_b1, mlp_W2, mlp_b2, mu_W, mu_b, lv_W, lv_b,
           dec_W1, dec_b1, dec_W2, dec_b2):
    f32 = jnp.float32
    row = edge_index[0].astype(jnp.int32).reshape(NW, NCH, CH)
    col = edge_index[1].astype(jnp.int32).reshape(NW, NCH, CH)
    ones128 = jnp.ones((CH, W128), f32)
    zeros128 = jnp.zeros((256, W128), f32)

    degp = _sc_deg(col, ones128, zeros128)
    degp2 = degp.reshape(2, N, W128)

    nb = 8
    xw, xwpL, xwpR = pl.pallas_call(
        _tc_a_body,
        grid=(nb,),
        in_specs=[
            pl.BlockSpec((_BR, 256), lambda i: (i, 0)),
            pl.BlockSpec((256, HID), lambda i: (0, 0)),
            pl.BlockSpec((2, _BR, W128), lambda i: (0, i, 0)),
        ],
        out_specs=[
            pl.BlockSpec((_BR, HID), lambda i: (i, 0)),
            pl.BlockSpec((_BR, W128), lambda i: (i, 0)),
            pl.BlockSpec((_BR, W128), lambda i: (i, 0)),
        ],
        out_shape=[
            jax.ShapeDtypeStruct((N, HID), f32),
            jax.ShapeDtypeStruct((N, W128), f32),
            jax.ShapeDtypeStruct((N, W128), f32),
        ],
    )(x, gcn_W, degp2)

    aggp = _sc_gcn(xwpL, xwpR, row, col, zeros128).reshape(2, 2, N, W128)

    t_shape = jax.ShapeDtypeStruct((N, W128), f32)
    t_spec = pl.BlockSpec((_BR, W128), lambda i: (i, 0))
    outs_b = pl.pallas_call(
        _tc_b_body,
        grid=(nb,),
        in_specs=[
            pl.BlockSpec((_BR, HID), lambda i: (i, 0)),
            pl.BlockSpec((2, _BR, W128), lambda i: (0, i, 0)),
            pl.BlockSpec((2, 2, _BR, W128), lambda i: (0, 0, i, 0)),
            pl.BlockSpec((1, HID), lambda i: (0, 0)),
            pl.BlockSpec((HID, HEADS * HID), lambda i: (0, 0)),
            pl.BlockSpec((HEADS, HID), lambda i: (0, 0)),
            pl.BlockSpec((HEADS, HID), lambda i: (0, 0)),
        ],
        out_specs=[t_spec] * 8 + [
            pl.BlockSpec((_BR, HEADS), lambda i: (i, 0)),
            pl.BlockSpec((_BR, HEADS), lambda i: (i, 0)),
            pl.BlockSpec((1, HEADS), lambda i: (0, 0)),
        ],
        out_shape=[t_shape] * 8 + [
            jax.ShapeDtypeStruct((N, HEADS), f32),
            jax.ShapeDtypeStruct((N, HEADS), f32),
            jax.ShapeDtypeStruct((1, HEADS), f32),
        ],
    )(xw, degp2, aggp, gcn_b.reshape(1, HID), gat_W, gat_att_src, gat_att_dst)
    tbls = outs_b[:8]
    asrc, adst, amax = outs_b[8:]

    atab, pself = pl.pallas_call(
        _tc_b2_body,
        out_shape=[
            jax.ShapeDtypeStruct((N, W128), f32),
            jax.ShapeDtypeStruct((N, HEADS), f32),
        ],
    )(asrc, adst, amax)

    outp = _sc_gat(*tbls, row, col, atab, zeros128)
    outp = outp.reshape(2, 9, N, W128)

    mu, logvar, hd = pl.pallas_call(
        _tc_c_body,
        grid=(nb,),
        in_specs=[
            pl.BlockSpec((2, 9, _BR, W128), lambda i: (0, 0, i, 0)),
        ] + [t_spec] * 8 + [
            pl.BlockSpec((_BR, HEADS), lambda i: (i, 0)),
            pl.BlockSpec((_BR, ZD), lambda i: (i, 0)),
            pl.BlockSpec((1, HID), lambda i: (0, 0)),
            pl.BlockSpec((HID, HID), lambda i: (0, 0)),
            pl.BlockSpec((1, HID), lambda i: (0, 0)),
            pl.BlockSpec((HID, HID), lambda i: (0, 0)),
            pl.BlockSpec((1, HID), lambda i: (0, 0)),
            pl.BlockSpec((HID, ZD), lambda i: (0, 0)),
            pl.BlockSpec((1, ZD), lambda i: (0, 0)),
            pl.BlockSpec((HID, ZD), lambda i: (0, 0)),
            pl.BlockSpec((1, ZD), lambda i: (0, 0)),
            pl.BlockSpec((ZD, HID), lambda i: (0, 0)),
            pl.BlockSpec((1, HID), lambda i: (0, 0)),
            pl.BlockSpec((HID, ZD), lambda i: (0, 0)),
            pl.BlockSpec((1, ZD), lambda i: (0, 0)),
        ],
        out_specs=[
            pl.BlockSpec((_BR, ZD), lambda i: (i, 0)),
            pl.BlockSpec((_BR, ZD), lambda i: (i, 0)),
            pl.BlockSpec((_BR, ZD), lambda i: (i, 0)),
        ],
        out_shape=[
            jax.ShapeDtypeStruct((N, ZD), f32),
            jax.ShapeDtypeStruct((N, ZD), f32),
            jax.ShapeDtypeStruct((N, ZD), f32),
        ],
    )(outp, *tbls, pself, eps,
      gat_b.reshape(1, HID), mlp_W1, mlp_b1.reshape(1, HID),
      mlp_W2, mlp_b2.reshape(1, HID), mu_W, mu_b.reshape(1, ZD),
      lv_W, lv_b.reshape(1, ZD), dec_W1, dec_b1.reshape(1, HID),
      dec_W2, dec_b2.reshape(1, ZD))

    logits = pl.pallas_call(
        _tc_d_body,
        grid=(nb,),
        in_specs=[
            pl.BlockSpec((_BR, ZD), lambda i: (i, 0)),
            pl.BlockSpec((N, ZD), lambda i: (0, 0)),
        ],
        out_specs=pl.BlockSpec((_BR, N), lambda i: (i, 0)),
        out_shape=jax.ShapeDtypeStruct((N, N), f32),
    )(hd, hd)

    return logits, mu, logvar


# speedup vs baseline: 1.0161x; 1.0149x over previous
import functools

import jax
import jax.numpy as jnp
from jax import lax
from jax.experimental import pallas as pl
from jax.experimental.pallas import tpu as pltpu
from jax.experimental.pallas import tpu_sc as plsc

N = 4096
E = 65536
HID = 256
ZD = 64
HEADS = 4
NW = 32
EPW = E // NW
NCH = 16
CH = EPW // NCH
W128 = 128

_mesh = plsc.VectorSubcoreMesh(core_axis_name="c", subcore_axis_name="s")


@functools.partial(
    pl.kernel, mesh=_mesh,
    out_type=jax.ShapeDtypeStruct((2 * N, W128), jnp.float32),
    scratch_types=[
        pltpu.VMEM((NCH, CH), jnp.int32),
        pltpu.VMEM((CH, W128), jnp.float32),
        pltpu.VMEM_SHARED((N, W128), jnp.float32),
    ],
)
def _sc_deg(colr, ones, zeros, out, cidx, ones_v, acc):
    cid = lax.axis_index("c")
    sid = lax.axis_index("s")
    wid = cid * 16 + sid
    pltpu.sync_copy(zeros, acc.at[pl.ds(sid * 256, 256)])
    pltpu.sync_copy(colr.at[wid], cidx)
    pltpu.sync_copy(ones, ones_v)
    plsc.subcore_barrier()
    for j in range(NCH):
        pltpu.sync_copy(ones_v, acc.at[cidx.at[j]], add=True)
    plsc.subcore_barrier()
    pltpu.sync_copy(acc.at[pl.ds(sid * 256, 256)],
                    out.at[pl.ds(cid * N + sid * 256, 256)])


@functools.partial(
    pl.kernel, mesh=_mesh,
    out_type=jax.ShapeDtypeStruct((2 * 2 * N, W128), jnp.float32),
    scratch_types=[
        pltpu.VMEM((NCH, CH), jnp.int32),
        pltpu.VMEM((NCH, CH), jnp.int32),
        pltpu.VMEM((CH, W128), jnp.float32),
        pltpu.VMEM((CH, W128), jnp.float32),
        pltpu.VMEM((CH, W128), jnp.float32),
        pltpu.VMEM_SHARED((N, W128), jnp.float32),
        pltpu.SemaphoreType.DMA,
        pltpu.SemaphoreType.DMA,
        pltpu.SemaphoreType.DMA,
        pltpu.SemaphoreType.DMA,
        pltpu.SemaphoreType.DMA,
        pltpu.SemaphoreType.DMA,
    ],
)
def _sc_gcn(xwpL, xwpR, rowr, colr, zeros, out, ridx, cidx, pay0, pay1, pay2,
            acc, g0, g1, g2, s0, s1, s2):
    cid = lax.axis_index("c")
    sid = lax.axis_index("s")
    wid = cid * 16 + sid
    pltpu.sync_copy(zeros, acc.at[pl.ds(sid * 256, 256)])
    pltpu.sync_copy(rowr.at[wid], ridx)
    pltpu.sync_copy(colr.at[wid], cidx)
    plsc.subcore_barrier()
    pays = (pay0, pay1, pay2)
    gsems = (g0, g1, g2)
    ssems = (s0, s1, s2)
    sps = [None, None, None]
    for t, tab in enumerate((xwpL, xwpR)):
        cps = [None, None, None]
        cps[0] = pltpu.async_copy(tab.at[ridx.at[0]], pays[0], gsems[0])
        for j in range(NCH):
            b = j % 3
            if j + 1 < NCH:
                nb = (j + 1) % 3
                if sps[nb] is not None:
                    sps[nb].wait()
                    sps[nb] = None
                cps[nb] = pltpu.async_copy(tab.at[ridx.at[j + 1]], pays[nb], gsems[nb])
            cps[b].wait()
            sps[b] = pltpu.async_copy(pays[b], acc.at[cidx.at[j]], ssems[b],
                                      add=True)
        for b in range(3):
            if sps[b] is not None:
                sps[b].wait()
                sps[b] = None
        plsc.subcore_barrier()
        pltpu.sync_copy(acc.at[pl.ds(sid * 256, 256)],
                        out.at[pl.ds((cid * 2 + t) * N + sid * 256, 256)])
        if t == 0:
            pltpu.sync_copy(zeros, acc.at[pl.ds(sid * 256, 256)])
        plsc.subcore_barrier()


NCH2 = 32
CH2 = EPW // NCH2


@functools.partial(
    pl.kernel, mesh=_mesh,
    out_type=jax.ShapeDtypeStruct((2 * 9 * N, W128), jnp.float32),
    scratch_types=[
        pltpu.VMEM((NCH2, CH2), jnp.int32),
        pltpu.VMEM((NCH2, CH2), jnp.int32),
        pltpu.VMEM((CH2, W128), jnp.float32),
        pltpu.VMEM((CH2, W128), jnp.float32),
        pltpu.VMEM((HEADS * N,), jnp.float32),
        pltpu.VMEM((HEADS * N,), jnp.float32),
        pltpu.VMEM((16,), jnp.float32),
        pltpu.VMEM((NCH2, CH2 * 16), jnp.float32),
        pltpu.VMEM_SHARED((N, W128), jnp.float32),
        pltpu.SemaphoreType.DMA,
        pltpu.SemaphoreType.DMA,
        pltpu.SemaphoreType.DMA,
        pltpu.SemaphoreType.DMA,
    ],
)
def _sc_gat(t0, t1, t2, t3, t4, t5, t6, t7, rowr, colr, asrcf, adstf, amaxh,
            zeros, out, ridx, cidx, pay0, pay1, af, df, axv, pbuf, acc,
            g0, g1, s0, s1):
    cid = lax.axis_index("c")
    sid = lax.axis_index("s")
    wid = cid * 16 + sid
    pltpu.sync_copy(zeros, acc.at[pl.ds(sid * 256, 256)])
    pltpu.sync_copy(rowr.at[wid], ridx)
    pltpu.sync_copy(colr.at[wid], cidx)
    pltpu.sync_copy(asrcf, af)
    pltpu.sync_copy(adstf, df)
    pltpu.sync_copy(amaxh, axv)

    def wait_gather(pay, sem):
        pltpu.make_async_copy(t0.at[ridx.at[0]], pay, sem).wait()

    def wait_scatter(pay, sem):
        pltpu.make_async_copy(pay, acc.at[cidx.at[0]], sem).wait()

    amaxv = axv[...]

    def p1_body(j, _):
        def g_body(g, _):
            rv = ridx[j, pl.ds(g * 16, 16)]
            cv = cidx[j, pl.ds(g * 16, 16)]
            for l in range(16):
                r = rv[l]
                c = cv[l]
                sv = af[pl.ds(r * 4, 16)]
                dv = df[pl.ds(c * 4, 16)]
                m = amaxv + dv
                m = jnp.maximum(m, 0.0) + 0.2 * jnp.minimum(m, 0.0)
                ev = sv + dv
                ev = jnp.maximum(ev, 0.0) + 0.2 * jnp.minimum(ev, 0.0)
                e = g * 16 + l
                pbuf[j, pl.ds(e * 16, 16)] = jnp.exp(ev - m)
            return 0

        lax.fori_loop(0, CH2 // 16, g_body, 0)
        return 0

    lax.fori_loop(0, NCH2, p1_body, 0)

    plsc.subcore_barrier()

    tabs = (t0, t1, t2, t3, t4, t5, t6, t7)
    for t in range(8):
        h = t // 2
        tab = tabs[t]
        pltpu.async_copy(tab.at[ridx.at[0]], pay0, g0)

        def scale_scatter(pay, gsem, ssem, j):
            wait_gather(pay, gsem)

            def s_body(e, _):
                ps = pbuf[j, pl.ds(e * 16, 16)][h]
                for c in range(W128 // 16):
                    pay[e, pl.ds(c * 16, 16)] = pay[e, pl.ds(c * 16, 16)] * ps
                return 0

            lax.fori_loop(0, CH2, s_body, 0)
            pltpu.async_copy(pay, acc.at[cidx.at[j]], ssem, add=True)

        def j_body(j, _):
            even = j % 2 == 0

            @pl.when(jnp.logical_and(j > 0, j < NCH2 - 1))
            def _():
                @pl.when(even)
                def _():
                    @pl.when(j > 1)
                    def _():
                        wait_scatter(pay1, s1)
                    pltpu.async_copy(tab.at[ridx.at[j + 1]], pay1, g1)

                @pl.when(jnp.logical_not(even))
                def _():
                    wait_scatter(pay0, s0)
                    pltpu.async_copy(tab.at[ridx.at[j + 1]], pay0, g0)

            @pl.when(j == 0)
            def _():
                pltpu.async_copy(tab.at[ridx.at[1]], pay1, g1)

            @pl.when(even)
            def _():
                scale_scatter(pay0, g0, s0, j)

            @pl.when(jnp.logical_not(even))
            def _():
                scale_scatter(pay1, g1, s1, j)

            return 0

        lax.fori_loop(0, NCH2, j_body, 0)
        wait_scatter(pay0, s0)
        wait_scatter(pay1, s1)
        plsc.subcore_barrier()
        pltpu.sync_copy(acc.at[pl.ds(sid * 256, 256)],
                        out.at[pl.ds((cid * 9 + t) * N + sid * 256, 256)])
        pltpu.sync_copy(zeros, acc.at[pl.ds(sid * 256, 256)])
        plsc.subcore_barrier()

    def p3_body(j, _):
        even = j % 2 == 0

        def build_scatter(pay, ssem):
            def d_body(e, _):
                pv = pbuf[j, pl.ds(e * 16, 16)]
                for c in range(W128 // 16):
                    pay[e, pl.ds(c * 16, 16)] = pv
                return 0

            lax.fori_loop(0, CH2, d_body, 0)
            pltpu.async_copy(pay, acc.at[cidx.at[j]], ssem, add=True)

        @pl.when(even)
        def _():
            @pl.when(j > 1)
            def _():
                wait_scatter(pay0, s0)
            build_scatter(pay0, s0)

        @pl.when(jnp.logical_not(even))
        def _():
            @pl.when(j > 1)
            def _():
                wait_scatter(pay1, s1)
            build_scatter(pay1, s1)

        return 0

    lax.fori_loop(0, NCH2, p3_body, 0)
    wait_scatter(pay0, s0)
    wait_scatter(pay1, s1)
    plsc.subcore_barrier()
    pltpu.sync_copy(acc.at[pl.ds(sid * 256, 256)],
                    out.at[pl.ds((cid * 9 + 8) * N + sid * 256, 256)])


_BR = 512


def _tc_a_body(x_ref, w_ref, degp_ref, xw_ref, xwpL_ref, xwpR_ref):
    xw = jnp.dot(x_ref[...], w_ref[...], preferred_element_type=jnp.float32)
    deg = degp_ref[0, :, 0] + degp_ref[1, :, 0] + 1.0
    dinv = lax.rsqrt(deg)
    xw_ref[...] = xw
    xwp = xw * dinv[:, None]
    xwpL_ref[...] = xwp[:, :W128]
    xwpR_ref[...] = xwp[:, W128:]


def _tc_b_body(xw_ref, degp_ref, aggp_ref, gcnb_ref, gatW_ref, atts_ref, attd_ref,
               t0_ref, t1_ref, t2_ref, t3_ref, t4_ref, t5_ref, t6_ref, t7_ref,
               asrc_ref, adst_ref, amax_ref):
    i = pl.program_id(0)
    deg = degp_ref[0, :, 0] + degp_ref[1, :, 0] + 1.0
    dinv = lax.rsqrt(deg)
    agghalves = aggp_ref[0] + aggp_ref[1]
    agg = jnp.concatenate([agghalves[0], agghalves[1]], axis=-1)
    xw = xw_ref[...]
    h = jax.nn.relu(dinv[:, None] * agg + (dinv * dinv)[:, None] * xw + gcnb_ref[...])
    xw4 = jnp.dot(h, gatW_ref[...], preferred_element_type=jnp.float32)
    trefs = (t0_ref, t1_ref, t2_ref, t3_ref, t4_ref, t5_ref, t6_ref, t7_ref)
    for t in range(8):
        trefs[t][...] = xw4[:, t * W128:(t + 1) * W128]
    xw4r = xw4.reshape(_BR, HEADS, HID)
    asrc = jnp.sum(xw4r * atts_ref[...][None], axis=-1)
    adst = jnp.sum(xw4r * attd_ref[...][None], axis=-1)
    asrc_ref[...] = asrc
    adst_ref[...] = adst

    @pl.when(i == 0)
    def _():
        amax_ref[...] = jnp.full((1, HEADS), -jnp.inf, jnp.float32)

    amax_ref[...] = jnp.maximum(amax_ref[...], jnp.max(asrc, axis=0, keepdims=True))


def _lrelu(x):
    return jnp.maximum(x, 0.0) + 0.2 * jnp.minimum(x, 0.0)


def _tc_b2_body(asrc_ref, adst_ref, amax_ref, pself_ref):
    asrc = asrc_ref[...]
    adst = adst_ref[...]
    amax = amax_ref[...]
    m = _lrelu(amax + adst)
    eself = _lrelu(asrc + adst)
    pself_ref[...] = jnp.exp(eself - m)


def _tc_c_body(outp_ref, t0_ref, t1_ref, t2_ref, t3_ref, t4_ref, t5_ref,
               t6_ref, t7_ref, pself_ref, eps_ref,
               gatb_ref, w1_ref, b1_ref, w2_ref, b2_ref, muW_ref, mub_ref,
               lvW_ref, lvb_ref, dW1_ref, db1_ref, dW2_ref, db2_ref,
               mu_ref, lv_ref, hd_ref):
    pre = outp_ref[0] + outp_ref[1]
    pself = pself_ref[...]
    trefs = (t0_ref, t1_ref, t2_ref, t3_ref, t4_ref, t5_ref, t6_ref, t7_ref)
    acc = jnp.zeros((_BR, HID), jnp.float32)
    for hh in range(HEADS):
        feat = jnp.concatenate([pre[2 * hh], pre[2 * hh + 1]], axis=-1)
        xw4h = jnp.concatenate([trefs[2 * hh][...], trefs[2 * hh + 1][...]], axis=-1)
        ph = pself[:, hh][:, None]
        feat = feat + ph * xw4h
        s = pre[8][:, hh][:, None] + ph
        acc = acc + feat / (s + 1e-16)
    hg = jax.nn.relu(acc * (1.0 / HEADS) + gatb_ref[...])
    h2 = jax.nn.relu(jnp.dot(hg, w1_ref[...], preferred_element_type=jnp.float32) + b1_ref[...])
    h3 = jax.nn.relu(jnp.dot(h2, w2_ref[...], preferred_element_type=jnp.float32) + b2_ref[...])
    mu = jnp.dot(h3, muW_ref[...], preferred_element_type=jnp.float32) + mub_ref[...]
    logvar = jnp.dot(h3, lvW_ref[...], preferred_element_type=jnp.float32) + lvb_ref[...]
    lvc = jnp.clip(logvar, -10.0, 10.0)
    z = mu + eps_ref[...] * jnp.exp(0.5 * lvc)
    hd1 = jax.nn.relu(jnp.dot(z, dW1_ref[...], preferred_element_type=jnp.float32) + db1_ref[...])
    hd = jnp.dot(hd1, dW2_ref[...], preferred_element_type=jnp.float32) + db2_ref[...] + z
    mu_ref[...] = mu
    lv_ref[...] = logvar
    hd_ref[...] = hd


def _tc_d_body(hdb_ref, hdf_ref, out_ref):
    out_ref[...] = lax.dot_general(
        hdb_ref[...], hdf_ref[...], (((1,), (1,)), ((), ())),
        preferred_element_type=jnp.float32)


def kernel(x, edge_index, eps, gcn_W, gcn_b, gat_W, gat_att_src, gat_att_dst,
           gat_b, mlp_W1, mlp_b1, mlp_W2, mlp_b2, mu_W, mu_b, lv_W, lv_b,
           dec_W1, dec_b1, dec_W2, dec_b2):
    f32 = jnp.float32
    row = edge_index[0].astype(jnp.int32).reshape(NW, NCH, CH)
    col = edge_index[1].astype(jnp.int32).reshape(NW, NCH, CH)
    ones128 = jnp.ones((CH, W128), f32)
    zeros128 = jnp.zeros((256, W128), f32)

    degp = _sc_deg(col, ones128, zeros128)
    degp2 = degp.reshape(2, N, W128)

    nb = 8
    xw, xwpL, xwpR = pl.pallas_call(
        _tc_a_body,
        grid=(nb,),
        in_specs=[
            pl.BlockSpec((_BR, 256), lambda i: (i, 0)),
            pl.BlockSpec((256, HID), lambda i: (0, 0)),
            pl.BlockSpec((2, _BR, W128), lambda i: (0, i, 0)),
        ],
        out_specs=[
            pl.BlockSpec((_BR, HID), lambda i: (i, 0)),
            pl.BlockSpec((_BR, W128), lambda i: (i, 0)),
            pl.BlockSpec((_BR, W128), lambda i: (i, 0)),
        ],
        out_shape=[
            jax.ShapeDtypeStruct((N, HID), f32),
            jax.ShapeDtypeStruct((N, W128), f32),
            jax.ShapeDtypeStruct((N, W128), f32),
        ],
    )(x, gcn_W, degp2)

    aggp = _sc_gcn(xwpL, xwpR, row, col, zeros128).reshape(2, 2, N, W128)

    t_shape = jax.ShapeDtypeStruct((N, W128), f32)
    t_spec = pl.BlockSpec((_BR, W128), lambda i: (i, 0))
    outs_b = pl.pallas_call(
        _tc_b_body,
        grid=(nb,),
        in_specs=[
            pl.BlockSpec((_BR, HID), lambda i: (i, 0)),
            pl.BlockSpec((2, _BR, W128), lambda i: (0, i, 0)),
            pl.BlockSpec((2, 2, _BR, W128), lambda i: (0, 0, i, 0)),
            pl.BlockSpec((1, HID), lambda i: (0, 0)),
            pl.BlockSpec((HID, HEADS * HID), lambda i: (0, 0)),
            pl.BlockSpec((HEADS, HID), lambda i: (0, 0)),
            pl.BlockSpec((HEADS, HID), lambda i: (0, 0)),
        ],
        out_specs=[t_spec] * 8 + [
            pl.BlockSpec((_BR, HEADS), lambda i: (i, 0)),
            pl.BlockSpec((_BR, HEADS), lambda i: (i, 0)),
            pl.BlockSpec((1, HEADS), lambda i: (0, 0)),
        ],
        out_shape=[t_shape] * 8 + [
            jax.ShapeDtypeStruct((N, HEADS), f32),
            jax.ShapeDtypeStruct((N, HEADS), f32),
            jax.ShapeDtypeStruct((1, HEADS), f32),
        ],
    )(xw, degp2, aggp, gcn_b.reshape(1, HID), gat_W, gat_att_src, gat_att_dst)
    tbls = outs_b[:8]
    asrc, adst, amax = outs_b[8:]

    pself = pl.pallas_call(
        _tc_b2_body,
        out_shape=jax.ShapeDtypeStruct((N, HEADS), f32),
    )(asrc, adst, amax)

    rowg = edge_index[0].astype(jnp.int32).reshape(NW, NCH2, CH2)
    colg = edge_index[1].astype(jnp.int32).reshape(NW, NCH2, CH2)
    amax16 = jnp.tile(amax, (1, 4)).reshape(16)
    outp = _sc_gat(*tbls, rowg, colg, asrc.reshape(N * HEADS),
                   adst.reshape(N * HEADS), amax16, zeros128)
    outp = outp.reshape(2, 9, N, W128)

    mu, logvar, hd = pl.pallas_call(
        _tc_c_body,
        grid=(nb,),
        in_specs=[
            pl.BlockSpec((2, 9, _BR, W128), lambda i: (0, 0, i, 0)),
        ] + [t_spec] * 8 + [
            pl.BlockSpec((_BR, HEADS), lambda i: (i, 0)),
            pl.BlockSpec((_BR, ZD), lambda i: (i, 0)),
            pl.BlockSpec((1, HID), lambda i: (0, 0)),
            pl.BlockSpec((HID, HID), lambda i: (0, 0)),
            pl.BlockSpec((1, HID), lambda i: (0, 0)),
            pl.BlockSpec((HID, HID), lambda i: (0, 0)),
            pl.BlockSpec((1, HID), lambda i: (0, 0)),
            pl.BlockSpec((HID, ZD), lambda i: (0, 0)),
            pl.BlockSpec((1, ZD), lambda i: (0, 0)),
            pl.BlockSpec((HID, ZD), lambda i: (0, 0)),
            pl.BlockSpec((1, ZD), lambda i: (0, 0)),
            pl.BlockSpec((ZD, HID), lambda i: (0, 0)),
            pl.BlockSpec((1, HID), lambda i: (0, 0)),
            pl.BlockSpec((HID, ZD), lambda i: (0, 0)),
            pl.BlockSpec((1, ZD), lambda i: (0, 0)),
        ],
        out_specs=[
            pl.BlockSpec((_BR, ZD), lambda i: (i, 0)),
            pl.BlockSpec((_BR, ZD), lambda i: (i, 0)),
            pl.BlockSpec((_BR, ZD), lambda i: (i, 0)),
        ],
        out_shape=[
            jax.ShapeDtypeStruct((N, ZD), f32),
            jax.ShapeDtypeStruct((N, ZD), f32),
            jax.ShapeDtypeStruct((N, ZD), f32),
        ],
    )(outp, *tbls, pself, eps,
      gat_b.reshape(1, HID), mlp_W1, mlp_b1.reshape(1, HID),
      mlp_W2, mlp_b2.reshape(1, HID), mu_W, mu_b.reshape(1, ZD),
      lv_W, lv_b.reshape(1, ZD), dec_W1, dec_b1.reshape(1, HID),
      dec_W2, dec_b2.reshape(1, ZD))

    logits = pl.pallas_call(
        _tc_d_body,
        grid=(nb,),
        in_specs=[
            pl.BlockSpec((_BR, ZD), lambda i: (i, 0)),
            pl.BlockSpec((N, ZD), lambda i: (0, 0)),
        ],
        out_specs=pl.BlockSpec((_BR, N), lambda i: (i, 0)),
        out_shape=jax.ShapeDtypeStruct((N, N), f32),
    )(hd, hd)

    return logits, mu, logvar
